# baseline (speedup 1.0000x reference)
import functools

import jax
import jax.numpy as jnp
from jax import lax
from jax.experimental import pallas as pl
from jax.experimental.pallas import tpu as pltpu
from jax.experimental.pallas import tpu_sc as plsc

N = 320000
D = 128
CD = D // 16
G_SEGS = 512
NC = 2
NS = 16
NW = NC * NS
ROWS_PER_W = N // NW
CHUNK = 80
NBUF = 5
NGROUP = ROWS_PER_W // (NBUF * CHUNK)
NCHUNK = ROWS_PER_W // CHUNK
SEG_PER_TILE = G_SEGS // NS
SLAB = (G_SEGS + 1) * D
TRASH = G_SEGS
ZROWS = 64
RED = SEG_PER_TILE * D


def _segment_sum_sc(x, batch_r):
    mesh = plsc.VectorSubcoreMesh(core_axis_name="c", subcore_axis_name="s")

    @functools.partial(
        pl.kernel,
        out_type=jax.ShapeDtypeStruct((NC, G_SEGS * D), jnp.float32),
        mesh=mesh,
        scratch_types=(
            [pltpu.VMEM((CHUNK * D,), jnp.float32) for _ in range(NBUF)]
            + [pltpu.VMEM((ZROWS * D,), jnp.float32),
               pltpu.VMEM((D,), jnp.float32),
               pltpu.VMEM((RED,), jnp.float32),
               pltpu.VMEM((RED,), jnp.float32),
               pltpu.VMEM((RED,), jnp.float32),
               pltpu.VMEM((ROWS_PER_W,), jnp.int32),
               pltpu.HBM((NW, SLAB), jnp.float32)]
            + [pltpu.SemaphoreType.DMA for _ in range(NBUF + 3)]
        ),
    )
    def k(x_hbm, b_hbm, out_hbm, *rest):
        bufs = rest[:NBUF]
        zbuf, stg, red, rb0, rb1, idx_v, arena = rest[NBUF:NBUF + 7]
        ls = rest[NBUF + 7:2 * NBUF + 7]
        rs0, rs1, msem = rest[2 * NBUF + 7:]

        c = lax.axis_index("c")
        s = lax.axis_index("s")
        wid = c * NS + s
        row0 = wid * ROWS_PER_W

        zro = jnp.zeros((16,), jnp.float32)

        pltpu.async_copy(b_hbm.at[wid], idx_v, msem).wait()

        @pl.loop(0, ZROWS * CD)
        def _zero_zbuf(r):
            zbuf[pl.ds(16 * r, 16)] = zro

        @pl.loop(0, SLAB // (ZROWS * D))
        def _zero_slab(t):
            pltpu.sync_copy(zbuf, arena.at[wid, pl.ds(t * ZROWS * D, ZROWS * D)])

        pltpu.sync_copy(zbuf.at[pl.ds(0, D)],
                        arena.at[wid, pl.ds(G_SEGS * D, D)])

        def load(j, b):
            return pltpu.async_copy(
                x_hbm.at[pl.ds((row0 + j * CHUNK) * D, CHUNK * D)],
                bufs[b], ls[b])

        def wait_load(b):
            pltpu.make_async_copy(
                x_hbm.at[pl.ds(0, CHUNK * D)], bufs[b], ls[b]).wait()

        for b in range(NBUF):
            load(b, b)

        def flush(accs, cur):
            for cc in range(CD):
                stg[pl.ds(16 * cc, 16)] = accs[cc]
            pltpu.sync_copy(stg, arena.at[wid, pl.ds(cur * D, D)])

        def run_group(g, carry):
            for b in range(NBUF):
                jc = g * NBUF + b
                wait_load(b)

                def blk_step(t, cr, _b=b, _jc=jc):
                    accs, cur = cr
                    seg_v = idx_v[pl.ds(_jc * CHUNK + t * 16, 16)]
                    for r in range(16):
                        seg = seg_v[r]
                        ch = seg != cur

                        @pl.when(ch)
                        def _flush(_a=accs, _c=cur):
                            flush(_a, _c)

                        accs = tuple(
                            jnp.where(ch, zro, accs[cc])
                            + bufs[_b][pl.ds((t * 16 + r) * D + 16 * cc, 16)]
                            for cc in range(CD))
                        cur = seg
                    return (accs, cur)

                carry = pl.loop(0, CHUNK // 16, init_carry=carry)(blk_step)

                @pl.when(jc + NBUF < NCHUNK)
                def _next_load(_b=b, _jc=jc):
                    load(_jc + NBUF, _b)

            return carry

        init = (tuple(zro for _ in range(CD)), jnp.int32(TRASH))
        accs, cur = pl.loop(0, NGROUP, init_carry=init)(run_group)
        flush(accs, cur)

        plsc.subcore_barrier()
        woff = s * RED
        rbs = (rb0, rb1)
        rss = (rs0, rs1)

        def rload(w, q):
            return pltpu.async_copy(
                arena.at[c * NS + w, pl.ds(woff, RED)], rbs[q], rss[q])

        def wait_rload(q):
            pltpu.make_async_copy(
                arena.at[0, pl.ds(0, RED)], rbs[q], rss[q]).wait()

        rload(0, 0)
        rload(1, 1)
        wait_rload(0)

        @pl.loop(0, RED // 16)
        def _init_red(r):
            red[pl.ds(16 * r, 16)] = rb0[pl.ds(16 * r, 16)]

        rload(2, 0)

        def red_pair(p, _):
            for q in range(2):
                w = 2 * p + 1 + q
                bq = (1 + q) % 2

                @pl.when(w < NS)
                def _acc(_w=w, _bq=bq):
                    wait_rload(_bq)

                    @pl.loop(0, RED // 16)
                    def _add(r):
                        red[pl.ds(16 * r, 16)] = (
                            red[pl.ds(16 * r, 16)]
                            + rbs[_bq][pl.ds(16 * r, 16)])

                    @pl.when(_w + 2 < NS)
                    def _next():
                        rload(_w + 2, _bq)

            return _

        pl.loop(0, NS // 2, init_carry=0)(red_pair)

        pltpu.async_copy(red, out_hbm.at[c, pl.ds(woff, RED)], msem).wait()

    return k(x, batch_r)


def kernel(input, batch, num_graphs):
    partials = _segment_sum_sc(input.reshape(-1),
                               batch.reshape(NW, ROWS_PER_W))
    out = (partials[0] + partials[1]).reshape(G_SEGS, D)
    return out + (jnp.asarray(num_graphs) - G_SEGS).astype(out.dtype)


# speedup vs baseline: 1.4584x; 1.4584x over previous
import functools

import jax
import jax.numpy as jnp
from jax import lax
from jax.experimental import pallas as pl
from jax.experimental.pallas import tpu as pltpu
from jax.experimental.pallas import tpu_sc as plsc

N = 320000
D = 128
G_SEGS = 512
NC = 2
NS = 16
NW = NC * NS
ROWS_PER_W = N // NW
CHUNK = 40
NCHUNK = ROWS_PER_W // CHUNK
NBUF = 10
NGROUP = NCHUNK // NBUF
SEG_PER_TILE = G_SEGS // NS


def _segment_sum_sc(x, batch_r):
    mesh = plsc.VectorSubcoreMesh(core_axis_name="c", subcore_axis_name="s")

    @functools.partial(
        pl.kernel,
        out_type=jax.ShapeDtypeStruct((NC, G_SEGS, D), jnp.float32),
        mesh=mesh,
        scratch_types=(
            [pltpu.VMEM((NCHUNK, CHUNK), jnp.int32)]
            + [pltpu.VMEM((CHUNK, D), jnp.float32) for _ in range(NBUF)]
            + [pltpu.VMEM((SEG_PER_TILE, D), jnp.float32),
               pltpu.VMEM_SHARED((G_SEGS, D), jnp.float32)]
            + [pltpu.SemaphoreType.DMA for _ in range(2 * NBUF + 1)]
        ),
    )
    def k(x_hbm, b_hbm, out_hbm, idx_v, *rest):
        bufs = rest[:NBUF]
        zbuf, acc = rest[NBUF], rest[NBUF + 1]
        ls = rest[NBUF + 2:2 * NBUF + 2]
        ws = rest[2 * NBUF + 2:3 * NBUF + 2]
        msem = rest[3 * NBUF + 2]
        c = lax.axis_index("c")
        s = lax.axis_index("s")
        wid = c * NS + s
        row0 = wid * ROWS_PER_W

        pltpu.async_copy(b_hbm.at[wid], idx_v, msem).wait()

        zero = jnp.zeros((16,), jnp.float32)

        @pl.loop(0, SEG_PER_TILE)
        def _zero_rows(r):
            for cc in range(D // 16):
                zbuf[r, pl.ds(cc * 16, 16)] = zero

        pltpu.sync_copy(zbuf, acc.at[pl.ds(s * SEG_PER_TILE, SEG_PER_TILE)])
        plsc.subcore_barrier()

        def load(j, b):
            return pltpu.async_copy(
                x_hbm.at[pl.ds(row0 + j * CHUNK, CHUNK)], bufs[b], ls[b])

        def wait_load(b):
            pltpu.make_async_copy(
                x_hbm.at[pl.ds(0, CHUNK)], bufs[b], ls[b]).wait()

        def scat(j, b):
            return pltpu.async_copy(bufs[b], acc.at[idx_v.at[j]], ws[b],
                                    add=True)

        for b in range(NBUF):
            load(b, b)

        @pl.loop(0, NGROUP)
        def _group(g):
            j0 = g * NBUF
            scats = []
            for b in range(NBUF):
                wait_load(b)
                scats.append(scat(j0 + b, b))
            for b in range(NBUF):
                scats[b].wait()

                @pl.when(g + 1 < NGROUP)
                def _next_load():
                    load(j0 + NBUF + b, b)

        plsc.subcore_barrier()
        pltpu.async_copy(
            acc.at[pl.ds(s * SEG_PER_TILE, SEG_PER_TILE)],
            out_hbm.at[c, pl.ds(s * SEG_PER_TILE, SEG_PER_TILE)],
            msem).wait()

    return k(x, batch_r)


def kernel(input, batch, num_graphs):
    partials = _segment_sum_sc(input, batch.reshape(NW, NCHUNK, CHUNK))
    out = partials[0] + partials[1]
    return out + (jnp.asarray(num_graphs) - G_SEGS).astype(out.dtype)


# speedup vs baseline: 1.4594x; 1.0007x over previous
import functools

import jax
import jax.numpy as jnp
from jax import lax
from jax.experimental import pallas as pl
from jax.experimental.pallas import tpu as pltpu
from jax.experimental.pallas import tpu_sc as plsc

N = 320000
D = 128
G_SEGS = 512
NC = 2
NS = 16
NW = NC * NS
ROWS_PER_W = N // NW
CHUNK = 80
NCHUNK = ROWS_PER_W // CHUNK
NBUF = 10
NGROUP = NCHUNK // NBUF
SEG_PER_TILE = G_SEGS // NS


def _segment_sum_sc(x, batch_r):
    mesh = plsc.VectorSubcoreMesh(core_axis_name="c", subcore_axis_name="s")

    @functools.partial(
        pl.kernel,
        out_type=jax.ShapeDtypeStruct((NC, G_SEGS, D), jnp.float32),
        mesh=mesh,
        scratch_types=(
            [pltpu.VMEM((NCHUNK, CHUNK), jnp.int32)]
            + [pltpu.VMEM((CHUNK, D), jnp.float32) for _ in range(NBUF)]
            + [pltpu.VMEM((SEG_PER_TILE, D), jnp.float32),
               pltpu.VMEM_SHARED((G_SEGS, D), jnp.float32)]
            + [pltpu.SemaphoreType.DMA for _ in range(2 * NBUF + 1)]
        ),
    )
    def k(x_hbm, b_hbm, out_hbm, idx_v, *rest):
        bufs = rest[:NBUF]
        zbuf, acc = rest[NBUF], rest[NBUF + 1]
        ls = rest[NBUF + 2:2 * NBUF + 2]
        ws = rest[2 * NBUF + 2:3 * NBUF + 2]
        msem = rest[3 * NBUF + 2]
        c = lax.axis_index("c")
        s = lax.axis_index("s")
        wid = c * NS + s
        row0 = wid * ROWS_PER_W

        pltpu.async_copy(b_hbm.at[wid], idx_v, msem).wait()

        zero = jnp.zeros((16,), jnp.float32)

        @pl.loop(0, SEG_PER_TILE)
        def _zero_rows(r):
            for cc in range(D // 16):
                zbuf[r, pl.ds(cc * 16, 16)] = zero

        pltpu.sync_copy(zbuf, acc.at[pl.ds(s * SEG_PER_TILE, SEG_PER_TILE)])
        plsc.subcore_barrier()

        def load(j, b):
            return pltpu.async_copy(
                x_hbm.at[pl.ds(row0 + j * CHUNK, CHUNK)], bufs[b], ls[b])

        def wait_load(b):
            pltpu.make_async_copy(
                x_hbm.at[pl.ds(0, CHUNK)], bufs[b], ls[b]).wait()

        def scat(j, b):
            return pltpu.async_copy(bufs[b], acc.at[idx_v.at[j]], ws[b],
                                    add=True)

        for b in range(NBUF):
            load(b, b)

        @pl.loop(0, NGROUP)
        def _group(g):
            j0 = g * NBUF
            scats = []
            for b in range(NBUF):
                wait_load(b)
                scats.append(scat(j0 + b, b))
            for b in range(NBUF):
                scats[b].wait()

                @pl.when(j0 + NBUF + b < NCHUNK)
                def _next_load():
                    load(j0 + NBUF + b, b)

        escats = []
        for b in range(NCHUNK - NGROUP * NBUF):
            wait_load(b)
            escats.append(scat(NGROUP * NBUF + b, b))
        for sc_ in escats:
            sc_.wait()

        plsc.subcore_barrier()
        pltpu.async_copy(
            acc.at[pl.ds(s * SEG_PER_TILE, SEG_PER_TILE)],
            out_hbm.at[c, pl.ds(s * SEG_PER_TILE, SEG_PER_TILE)],
            msem).wait()

    return k(x, batch_r)


def kernel(input, batch, num_graphs):
    partials = _segment_sum_sc(input, batch.reshape(NW, NCHUNK, CHUNK))
    out = partials[0] + partials[1]
    return out + (jnp.asarray(num_graphs) - G_SEGS).astype(out.dtype)
